# Initial kernel scaffold; baseline (speedup 1.0000x reference)
import jax
import jax.numpy as jnp
from jax.experimental import pallas as pl


def kernel(intent_id, table, ln_weight, ln_bias):
    raise NotImplementedError("write your pallas kernel here")


# speedup vs baseline: 1.3427x; 1.3427x over previous
import functools

import jax
import jax.numpy as jnp
from jax import lax
from jax.experimental import pallas as pl
from jax.experimental.pallas import tpu as pltpu
from jax.experimental.pallas import tpu_sc as plsc

_NC = 2
_NS = 16
_NW = _NC * _NS
_L = 16

_B = 16384
_D = 64
_KC = _D // _L
_BPW = _B // _NW
_CHUNK = 128
_NCHUNK = _BPW // _CHUNK
_UNROLL = 4


def _rsqrt(x):
    xi = lax.bitcast_convert_type(x, jnp.int32)
    y = lax.bitcast_convert_type(jnp.int32(0x5F3759DF) - (xi >> 1), jnp.float32)
    xh = x * jnp.float32(-0.5)
    for _ in range(3):
        y = y * (jnp.float32(1.5) + xh * y * y)
    return y


def _ln_kernel(idx_hbm, table_hbm, w_hbm, b_hbm, out_hbm,
               idx_v, rows_v, w_v, b_v, sem):
    wid = lax.axis_index("s") * _NC + lax.axis_index("c")
    base = wid * _BPW

    pltpu.sync_copy(idx_hbm.at[wid], idx_v)
    pltpu.sync_copy(w_hbm, w_v)
    pltpu.sync_copy(b_hbm, b_v)

    copies = [
        pltpu.async_copy(
            table_hbm.at[idx_v.at[c]],
            rows_v.at[pl.ds(c * _CHUNK, _CHUNK)],
            sem,
        )
        for c in range(_NCHUNK)
    ]
    for cp in copies:
        cp.wait()

    inv_d = jnp.float32(1.0 / _D)
    eps = jnp.float32(1e-5)
    wc = [w_v[pl.ds(k * _L, _L)] for k in range(_KC)]
    bc = [b_v[pl.ds(k * _L, _L)] for k in range(_KC)]

    def row_body(it, carry):
        for u in range(_UNROLL):
            r = it * _UNROLL + u
            c = [rows_v[r, pl.ds(k * _L, _L)] for k in range(_KC)]
            s = c[0] + c[1] + c[2] + c[3]
            q = c[0] * c[0] + c[1] * c[1] + c[2] * c[2] + c[3] * c[3]
            mean = jnp.sum(s) * inv_d
            var = jnp.sum(q) * inv_d - mean * mean
            rs = _rsqrt(var + eps)
            for k in range(_KC):
                rows_v[r, pl.ds(k * _L, _L)] = (c[k] - mean) * rs * wc[k] + bc[k]
        return carry

    lax.fori_loop(0, _BPW // _UNROLL, row_body, 0)

    pltpu.sync_copy(rows_v, out_hbm.at[pl.ds(base, _BPW)])


@jax.jit
def _run(idx3, table, ln_weight, ln_bias):
    mesh = plsc.VectorSubcoreMesh(core_axis_name="c", subcore_axis_name="s")
    f = functools.partial(
        pl.kernel,
        mesh=mesh,
        out_type=jax.ShapeDtypeStruct((_B, _D), jnp.float32),
        compiler_params=pltpu.CompilerParams(
            needs_layout_passes=False, use_tc_tiling_on_sc=False
        ),
        scratch_types=[
            pltpu.VMEM((_NCHUNK, _CHUNK), jnp.int32),
            pltpu.VMEM((_BPW, _D), jnp.float32),
            pltpu.VMEM((_D,), jnp.float32),
            pltpu.VMEM((_D,), jnp.float32),
            pltpu.SemaphoreType.DMA,
        ],
    )(_ln_kernel)
    return f(idx3, table, ln_weight, ln_bias)


def kernel(intent_id, table, ln_weight, ln_bias):
    idx3 = intent_id.astype(jnp.int32).reshape(_NW, _NCHUNK, _CHUNK)
    return _run(idx3, table, ln_weight, ln_bias)


# speedup vs baseline: 1.5284x; 1.1382x over previous
import functools

import jax
import jax.numpy as jnp
from jax import lax
from jax.experimental import pallas as pl
from jax.experimental.pallas import tpu as pltpu
from jax.experimental.pallas import tpu_sc as plsc

_NC = 2
_NS = 16
_NW = _NC * _NS
_L = 16

_B = 16384
_V = 1000
_D = 64
_KC = _D // _L
_BPW = _B // _NW
_CHUNK = 128
_NCHUNK = _BPW // _CHUNK
_TROWS = 64
_UNROLL = 4


def _rsqrt(x):
    xi = lax.bitcast_convert_type(x, jnp.int32)
    y = lax.bitcast_convert_type(jnp.int32(0x5F3759DF) - (xi >> 1), jnp.float32)
    xh = x * jnp.float32(-0.5)
    for _ in range(3):
        y = y * (jnp.float32(1.5) + xh * y * y)
    return y


def _ln_kernel(idx_hbm, table_hbm, w_hbm, b_hbm, out_hbm,
               idx_v, tbl_v, rows_v, w_v, b_v, nt_hbm, sem, sem_out):
    cid = lax.axis_index("c")
    sid = lax.axis_index("s")
    wid = sid * _NC + cid
    base = wid * _BPW

    start = jnp.minimum(sid * _TROWS, _V - _TROWS)
    pltpu.sync_copy(w_hbm, w_v)
    pltpu.sync_copy(b_hbm, b_v)
    pltpu.sync_copy(table_hbm.at[pl.ds(start, _TROWS)], tbl_v)
    pltpu.sync_copy(idx_hbm.at[wid], idx_v)

    inv_d = jnp.float32(1.0 / _D)
    eps = jnp.float32(1e-5)
    wc = [w_v[pl.ds(k * _L, _L)] for k in range(_KC)]
    bc = [b_v[pl.ds(k * _L, _L)] for k in range(_KC)]

    def row_body(it, carry):
        for u in range(_UNROLL):
            r = it * _UNROLL + u
            c = [tbl_v[r, pl.ds(k * _L, _L)] for k in range(_KC)]
            s = c[0] + c[1] + c[2] + c[3]
            q = c[0] * c[0] + c[1] * c[1] + c[2] * c[2] + c[3] * c[3]
            mean = jnp.sum(s) * inv_d
            var = jnp.sum(q) * inv_d - mean * mean
            rs = _rsqrt(var + eps)
            for k in range(_KC):
                tbl_v[r, pl.ds(k * _L, _L)] = (c[k] - mean) * rs * wc[k] + bc[k]
        return carry

    lax.fori_loop(0, _TROWS // _UNROLL, row_body, 0)

    pltpu.sync_copy(tbl_v, nt_hbm.at[cid].at[pl.ds(start, _TROWS)])
    plsc.subcore_barrier()

    gathers = [
        pltpu.async_copy(
            nt_hbm.at[cid].at[idx_v.at[c]],
            rows_v.at[pl.ds(c * _CHUNK, _CHUNK)],
            sem,
        )
        for c in range(_NCHUNK)
    ]
    writes = []
    for c in range(_NCHUNK):
        gathers[c].wait()
        writes.append(
            pltpu.async_copy(
                rows_v.at[pl.ds(c * _CHUNK, _CHUNK)],
                out_hbm.at[pl.ds(base + c * _CHUNK, _CHUNK)],
                sem_out,
            )
        )
    for wcp in writes:
        wcp.wait()


@jax.jit
def _run(idx3, table, ln_weight, ln_bias):
    mesh = plsc.VectorSubcoreMesh(core_axis_name="c", subcore_axis_name="s")
    f = functools.partial(
        pl.kernel,
        mesh=mesh,
        out_type=jax.ShapeDtypeStruct((_B, _D), jnp.float32),
        compiler_params=pltpu.CompilerParams(
            needs_layout_passes=False, use_tc_tiling_on_sc=False
        ),
        scratch_types=[
            pltpu.VMEM((_NCHUNK, _CHUNK), jnp.int32),
            pltpu.VMEM((_TROWS, _D), jnp.float32),
            pltpu.VMEM((_BPW, _D), jnp.float32),
            pltpu.VMEM((_D,), jnp.float32),
            pltpu.VMEM((_D,), jnp.float32),
            pltpu.HBM((_NC, _V, _D), jnp.float32),
            pltpu.SemaphoreType.DMA,
            pltpu.SemaphoreType.DMA,
        ],
    )(_ln_kernel)
    return f(idx3, table, ln_weight, ln_bias)


def kernel(intent_id, table, ln_weight, ln_bias):
    idx3 = intent_id.astype(jnp.int32).reshape(_NW, _NCHUNK, _CHUNK)
    return _run(idx3, table, ln_weight, ln_bias)
